# baseline (speedup 1.0000x reference)
import jax
import jax.numpy as jnp
from jax.experimental import pallas as pl

TOPK = 300
Q = 20000
NCLS = 80
R, C = 200, 100

INT_MIN = -(2**31)
BIG = 2**30


def _topk_kernel(bboxes_ref, logits_ref, bb_out_ref, lg_out_ref):
    logit = logits_ref[0]
    scores = jnp.max(logit.reshape(R, C, NCLS), axis=2)
    ikey = jax.lax.bitcast_convert_type(scores, jnp.int32)
    keys = ikey ^ jax.lax.shift_right_logical(
        jax.lax.shift_right_arithmetic(ikey, 31), 1
    )
    qiota = (
        jax.lax.broadcasted_iota(jnp.int32, (R, C), 0) * C
        + jax.lax.broadcasted_iota(jnp.int32, (R, C), 1)
    )

    def body(i, keys):
        m = jnp.max(keys)
        idx = jnp.min(jnp.where(keys == m, qiota, BIG))
        keys = jnp.where(qiota == idx, INT_MIN, keys)
        lg_out_ref[0, pl.ds(i, 1), :] = logits_ref[0, pl.ds(idx, 1), :]
        bb_out_ref[0, pl.ds(i, 1), :] = bboxes_ref[0, pl.ds(idx, 1), :]
        return keys

    jax.lax.fori_loop(0, TOPK, body, keys)


def kernel(bboxes, logits):
    B = bboxes.shape[0]
    bb_out, lg_out = pl.pallas_call(
        _topk_kernel,
        grid=(B,),
        in_specs=[
            pl.BlockSpec((1, Q, 4), lambda b: (b, 0, 0)),
            pl.BlockSpec((1, Q, NCLS), lambda b: (b, 0, 0)),
        ],
        out_specs=[
            pl.BlockSpec((1, TOPK, 4), lambda b: (b, 0, 0)),
            pl.BlockSpec((1, TOPK, NCLS), lambda b: (b, 0, 0)),
        ],
        out_shape=[
            jax.ShapeDtypeStruct((B, TOPK, 4), jnp.float32),
            jax.ShapeDtypeStruct((B, TOPK, NCLS), jnp.float32),
        ],
    )(bboxes, logits)
    return (bb_out, lg_out)
